# Initial kernel scaffold; baseline (speedup 1.0000x reference)
import jax
import jax.numpy as jnp
from jax.experimental import pallas as pl


def kernel(x, x_list_0, x_list_1, S_list_0, S_list_1, W0, W1, W2, W3, W4, W5, b0, b1, b2, b3, b4, b5, edge_index_list_0, edge_index_list_1, edge_index_list_2):
    raise NotImplementedError("write your pallas kernel here")


# speedup vs baseline: 8.1395x; 8.1395x over previous
import functools

import jax
import jax.numpy as jnp
from jax import lax
from jax.experimental import pallas as pl
from jax.experimental.pallas import tpu as pltpu
from jax.experimental.pallas import tpu_sc as plsc

D = 128
NC = 2
NS = 16
NW = NC * NS
L = 16
EB = 128


def _round_up(v, m):
  return (v + m - 1) // m * m


def _make_deg_kernel(n_pad, ew):
  mesh = plsc.VectorSubcoreMesh(core_axis_name="c", subcore_axis_name="s")

  @functools.partial(
      pl.kernel,
      out_type=jax.ShapeDtypeStruct((NW, n_pad), jnp.float32),
      mesh=mesh,
      scratch_types=[
          pltpu.VMEM((n_pad,), jnp.float32),
          pltpu.VMEM((ew,), jnp.int32),
      ],
      compiler_params=pltpu.CompilerParams(needs_layout_passes=False),
  )
  def deg_kernel(dst_hbm, out_hbm, deg_v, idx_v):
    c = lax.axis_index("c")
    s = lax.axis_index("s")
    w = c * NS + s

    zeros = jnp.zeros((L,), jnp.float32)

    def zero_body(i, _):
      deg_v[pl.ds(pl.multiple_of(i * L, L), L)] = zeros
      return 0

    lax.fori_loop(0, n_pad // L, zero_body, 0)

    pltpu.sync_copy(dst_hbm.at[pl.ds(w * ew, ew)], idx_v)

    ones = jnp.ones((L,), jnp.float32)

    def cnt_body(j, _):
      idx = idx_v[pl.ds(pl.multiple_of(j * L, L), L)]
      plsc.addupdate_scatter(deg_v, [idx], ones)
      return 0

    lax.fori_loop(0, ew // L, cnt_body, 0)

    pltpu.sync_copy(deg_v, out_hbm.at[w])

  return deg_kernel


def _make_agg_kernel(n_pad, ew):
  mesh = plsc.VectorSubcoreMesh(core_axis_name="c", subcore_axis_name="s")
  rpt = n_pad // NS
  nblk = ew // EB

  @functools.partial(
      pl.kernel,
      out_type=jax.ShapeDtypeStruct((NC, n_pad, D), jnp.float32),
      mesh=mesh,
      scratch_types=[
          pltpu.VMEM((EB,), jnp.int32),
          pltpu.VMEM((EB,), jnp.int32),
          pltpu.VMEM((EB, D), jnp.float32),
          pltpu.VMEM_SHARED((n_pad, D), jnp.float32),
          pltpu.SemaphoreType.DMA,
      ],
      compiler_params=pltpu.CompilerParams(needs_layout_passes=False),
  )
  def agg_kernel(y_hbm, src_hbm, dst_hbm, out_hbm, si_v, di_v, rows_v,
                 acc_sh, sem):
    c = lax.axis_index("c")
    s = lax.axis_index("s")
    w = c * NS + s

    zeros = jnp.zeros((L,), jnp.float32)

    def zero_body(i, _):
      def zrow(k, _):
        rows_v[i, pl.ds(pl.multiple_of(k * L, L), L)] = zeros
        return 0

      lax.fori_loop(0, D // L, zrow, 0)
      return 0

    lax.fori_loop(0, EB, zero_body, 0)

    base = s * rpt
    nfull = rpt // EB
    rem = rpt % EB
    for it in range(nfull):
      pltpu.sync_copy(rows_v, acc_sh.at[pl.ds(base + it * EB, EB)])
    if rem:
      pltpu.sync_copy(rows_v.at[pl.ds(0, rem)],
                      acc_sh.at[pl.ds(base + nfull * EB, rem)])
    plsc.subcore_barrier()

    def blk_body(bi, _):
      e0 = w * ew + bi * EB
      pltpu.sync_copy(src_hbm.at[pl.ds(e0, EB)], si_v)
      pltpu.sync_copy(dst_hbm.at[pl.ds(e0, EB)], di_v)
      pltpu.async_copy(y_hbm.at[si_v], rows_v, sem).wait()
      pltpu.sync_copy(rows_v, acc_sh.at[di_v], add=True)
      return 0

    lax.fori_loop(0, nblk, blk_body, 0)

    plsc.subcore_barrier()
    pltpu.sync_copy(acc_sh.at[pl.ds(base, rpt)],
                    out_hbm.at[c, pl.ds(base, rpt)])

  return agg_kernel


def _dinv_body(degp_ref, out_ref):
  s = jnp.sum(degp_ref[...], axis=0) + 1.0
  out_ref[...] = lax.rsqrt(s)[:, None]


def _dinv(degp, n_pad):
  return pl.pallas_call(
      _dinv_body,
      out_shape=jax.ShapeDtypeStruct((n_pad, 1), jnp.float32),
  )(degp)


def _y_body(bm, h_ref, dinv_ref, w_ref, o_ref):
  dinv = dinv_ref[...][:bm]
  o_ref[...] = jnp.dot(h_ref[...] * dinv, w_ref[...],
                       preferred_element_type=jnp.float32)


def _y_mm(h, dinv, w, bm):
  n = h.shape[0]
  n_pad = dinv.shape[0]
  bz = bm if n // bm > 1 else n_pad
  return pl.pallas_call(
      functools.partial(_y_body, bm),
      grid=(n // bm,),
      in_specs=[
          pl.BlockSpec((bm, D), lambda i: (i, 0)),
          pl.BlockSpec((bz, 1), lambda i: (i, 0)),
          pl.BlockSpec((D, D), lambda i: (0, 0)),
      ],
      out_specs=pl.BlockSpec((bm, D), lambda i: (i, 0)),
      out_shape=jax.ShapeDtypeStruct((n, D), jnp.float32),
  )(h, dinv, w)


def _mid_body(bm, zz_ref, y_ref, dinv_ref, b_ref, w_ref, o_ref):
  z = (zz_ref[0] + zz_ref[1])[:bm]
  dinv = dinv_ref[...][:bm]
  h1 = jnp.maximum((z + y_ref[...]) * dinv + b_ref[...], 0.0)
  o_ref[...] = jnp.dot(h1 * dinv, w_ref[...],
                       preferred_element_type=jnp.float32)


def _mid_mm(zz, y, dinv, b, w, bm):
  n = y.shape[0]
  n_pad = dinv.shape[0]
  bz = bm if n // bm > 1 else n_pad
  return pl.pallas_call(
      functools.partial(_mid_body, bm),
      grid=(n // bm,),
      in_specs=[
          pl.BlockSpec((NC, bz, D), lambda i: (0, i, 0)),
          pl.BlockSpec((bm, D), lambda i: (i, 0)),
          pl.BlockSpec((bz, 1), lambda i: (i, 0)),
          pl.BlockSpec((1, D), lambda i: (0, 0)),
          pl.BlockSpec((D, D), lambda i: (0, 0)),
      ],
      out_specs=pl.BlockSpec((bm, D), lambda i: (i, 0)),
      out_shape=jax.ShapeDtypeStruct((n, D), jnp.float32),
  )(zz, y, dinv, b, w)


def _fin_body(bm, relu, zz_ref, y_ref, dinv_ref, b_ref, o_ref):
  z = (zz_ref[0] + zz_ref[1])[:bm]
  dinv = dinv_ref[...][:bm]
  h2 = (z + y_ref[...]) * dinv + b_ref[...]
  if relu:
    h2 = jnp.maximum(h2, 0.0)
  o_ref[...] = h2


def _fin(zz, y, dinv, b, bm, relu):
  n = y.shape[0]
  n_pad = dinv.shape[0]
  bz = bm if n // bm > 1 else n_pad
  return pl.pallas_call(
      functools.partial(_fin_body, bm, relu),
      grid=(n // bm,),
      in_specs=[
          pl.BlockSpec((NC, bz, D), lambda i: (0, i, 0)),
          pl.BlockSpec((bm, D), lambda i: (i, 0)),
          pl.BlockSpec((bz, 1), lambda i: (i, 0)),
          pl.BlockSpec((1, D), lambda i: (0, 0)),
      ],
      out_specs=pl.BlockSpec((bm, D), lambda i: (i, 0)),
      out_shape=jax.ShapeDtypeStruct((n, D), jnp.float32),
  )(zz, y, dinv, b)


def _prolong_body(s_ref, h_ref, x_ref, o_ref):
  o_ref[...] = jnp.dot(s_ref[...], h_ref[...],
                       preferred_element_type=jnp.float32) + x_ref[...]


def _prolong(s_mat, h, x, bm):
  m, k = s_mat.shape
  return pl.pallas_call(
      _prolong_body,
      grid=(m // bm,),
      in_specs=[
          pl.BlockSpec((bm, k), lambda i: (i, 0)),
          pl.BlockSpec((k, D), lambda i: (0, 0)),
          pl.BlockSpec((bm, D), lambda i: (i, 0)),
      ],
      out_specs=pl.BlockSpec((bm, D), lambda i: (i, 0)),
      out_shape=jax.ShapeDtypeStruct((m, D), jnp.float32),
  )(s_mat, h, x)


def _pad_edges(edge_index, n_pad):
  src = edge_index[0]
  dst = edge_index[1]
  e = src.shape[0]
  e_pad = _round_up(e, NW * EB)
  pad = e_pad - e
  if pad:
    src = jnp.concatenate([src, jnp.zeros((pad,), jnp.int32)])
    dst = jnp.concatenate(
        [dst, jnp.full((pad,), n_pad - 1, jnp.int32)])
  return src, dst, e_pad


def _gcn_level(h, src, dst, e_pad, wa, ba, wb, bb, n, n_pad, bm, relu):
  ew = e_pad // NW
  degp = _make_deg_kernel(n_pad, ew)(dst)
  dinv = _dinv(degp, n_pad)
  agg = _make_agg_kernel(n_pad, ew)
  y0 = _y_mm(h, dinv, wa, bm)
  zz0 = agg(y0, src, dst)
  y1 = _mid_mm(zz0, y0, dinv, ba, wb, bm)
  zz1 = agg(y1, src, dst)
  return _fin(zz1, y1, dinv, bb, bm, relu)


def kernel(x, x_list_0, x_list_1, S_list_0, S_list_1,
           W0, W1, W2, W3, W4, W5, b0, b1, b2, b3, b4, b5,
           edge_index_list_0, edge_index_list_1, edge_index_list_2):
  n0, n1, n2 = x_list_0.shape[0], x_list_1.shape[0], x.shape[0]
  npm = NS * 8
  np0, np1, np2 = _round_up(n0, npm), _round_up(n1, npm), _round_up(n2, npm)

  b0r, b1r, b2r = b0[None, :], b1[None, :], b2[None, :]
  b3r, b4r, b5r = b3[None, :], b4[None, :], b5[None, :]

  s2, d2, ep2 = _pad_edges(edge_index_list_2, np2)
  s1, d1, ep1 = _pad_edges(edge_index_list_1, np1)
  s0, d0, ep0 = _pad_edges(edge_index_list_0, np0)

  h = _gcn_level(x, s2, d2, ep2, W0, b0r, W1, b1r, n2, np2, n2, True)
  h = _prolong(S_list_1, h, x_list_1, n1)
  h = _gcn_level(h, s1, d1, ep1, W2, b2r, W3, b3r, n1, np1, n1, True)
  h = _prolong(S_list_0, h, x_list_0, 1000)
  h = _gcn_level(h, s0, d0, ep0, W4, b4r, W5, b5r, n0, np0, 1000, False)
  return h
